# baseline (speedup 1.0000x reference)
import functools

import jax
import jax.numpy as jnp
from jax import lax
from jax.experimental import pallas as pl
from jax.experimental.pallas import tpu as pltpu
from jax.experimental.pallas import tpu_sc as plsc

NC = 2
NS = 16
NW = NC * NS

EPS = 1e-5
RPC = 2
GRPS = (0, 128)
GRPL = (128, 72)
BB = 16


def _make_gather(B, S, V, D, SP):
    rows_w = B // NW
    n_chunks = rows_w // RPC
    rpc16 = RPC * S * D // (8 * 128)
    mesh = plsc.VectorSubcoreMesh(
        core_axis_name="c", subcore_axis_name="s", num_cores=NC, num_subcores=NS
    )

    @functools.partial(
        pl.kernel,
        out_type=jax.ShapeDtypeStruct((B * S, D), jnp.float32),
        mesh=mesh,
        scratch_types=[
            pltpu.VMEM((2, RPC, SP), jnp.int32),
            pltpu.VMEM((2, RPC * S, D), jnp.float32),
            pltpu.SemaphoreType.DMA,
            pltpu.SemaphoreType.DMA,
        ],
        compiler_params=pltpu.CompilerParams(use_tc_tiling_on_sc=False),
    )
    def k(idx_hbm, table_hbm, out_hbm, idx_v, rows_v, sem_g, sem_o):
        wid = lax.axis_index("s") * NC + lax.axis_index("c")
        brow0 = wid * rows_w

        def start_gathers(g, b):
            brow = brow0 + g * RPC
            pltpu.sync_copy(idx_hbm.at[pl.ds(brow, RPC)], idx_v.at[b])
            for r in range(RPC):
                for o, n in zip(GRPS, GRPL):
                    pltpu.async_copy(
                        table_hbm.at[idx_v.at[b, r, pl.ds(o, n)]],
                        rows_v.at[b, pl.ds(r * S + o, n)],
                        sem_g,
                    )

        def wait_chunk(b, sem):
            pltpu.make_async_copy(
                out_hbm.at[pl.ds(0, RPC * S)], rows_v.at[b], sem
            ).wait()

        start_gathers(0, 0)

        def chunk_body(g, carry):
            b = g % 2
            wait_chunk(b, sem_g)

            @pl.when(g + 1 < n_chunks)
            def _():
                @pl.when(g >= 1)
                def _():
                    wait_chunk(1 - b, sem_o)
                start_gathers(g + 1, 1 - b)

            pltpu.async_copy(
                rows_v.at[b],
                out_hbm.at[pl.ds((brow0 + g * RPC) * S, RPC * S)],
                sem_o,
            )
            return carry

        lax.fori_loop(0, n_chunks, chunk_body, 0)
        wait_chunk(0, sem_o)
        wait_chunk(1, sem_o)

    return k


def _ln_body(e_ref, gb_ref, out_ref):
    nrow2 = e_ref.shape[0] * 8
    x = e_ref[...].reshape(nrow2, 128)
    gam = gb_ref[0, :]
    bet = gb_ref[1, :]
    inv_d = jnp.float32(1.0 / 64)
    s_lo = jnp.sum(x[:, :64], axis=1) * inv_d
    s_hi = jnp.sum(x[:, 64:], axis=1) * inv_d
    q_lo = jnp.sum(jnp.square(x[:, :64]), axis=1) * inv_d
    q_hi = jnp.sum(jnp.square(x[:, 64:]), axis=1) * inv_d
    r_lo = lax.rsqrt(q_lo - s_lo * s_lo + jnp.float32(EPS))[:, None]
    r_hi = lax.rsqrt(q_hi - s_hi * s_hi + jnp.float32(EPS))[:, None]
    even = (x[:, :64] - s_lo[:, None]) * r_lo * gam + bet
    odd = (x[:, 64:] - s_hi[:, None]) * r_hi * gam + bet
    out = jnp.stack([even, odd], axis=1).reshape(out_ref.shape)
    out_ref[...] = out


def _make_ln(B, S, D):
    grid = B // BB
    lines = BB * S * D // 1024

    return pl.pallas_call(
        _ln_body,
        grid=(grid,),
        in_specs=[
            pl.BlockSpec((lines, 8, 128), lambda i: (i, 0, 0)),
            pl.BlockSpec((2, D), lambda i: (0, 0)),
        ],
        out_specs=pl.BlockSpec((BB, S, D), lambda i: (i, 0, 0)),
        out_shape=jax.ShapeDtypeStruct((B, S, D), jnp.float32),
        compiler_params=pltpu.CompilerParams(
            dimension_semantics=("arbitrary",),
        ),
    )


def kernel(x, table, gamma, beta):
    B, S = x.shape
    V, D = table.shape
    SP = 256
    xp = jnp.pad(x.astype(jnp.int32), ((0, 0), (0, SP - S)))
    e = _make_gather(B, S, V, D, SP)(xp, table)
    e = e.reshape(B * S * D // 1024, 8, 128)
    gb = jnp.stack([gamma, beta]).astype(jnp.float32)
    return _make_ln(B, S, D)(e, gb)


# speedup vs baseline: 1.4212x; 1.4212x over previous
import functools

import jax
import jax.numpy as jnp
from jax import lax
from jax.experimental import pallas as pl
from jax.experimental.pallas import tpu as pltpu
from jax.experimental.pallas import tpu_sc as plsc

NC = 2
NS = 16
NW = NC * NS

EPS = 1e-5
RPC = 2
GRPS = (0, 128)
GRPL = (128, 72)
BBT = 128


def _make_gather(B, S, V):
    rows_w = B // NW
    n_chunks = rows_w // RPC
    qrows = rows_w // 4
    qchunks = n_chunks // 4
    mesh = plsc.VectorSubcoreMesh(
        core_axis_name="c", subcore_axis_name="s", num_cores=NC, num_subcores=NS
    )

    @functools.partial(
        pl.kernel,
        out_type=jax.ShapeDtypeStruct((B * S, 128), jnp.float32),
        mesh=mesh,
        scratch_types=[
            pltpu.VMEM((2, qrows, S), jnp.int32),
            pltpu.VMEM((2, RPC * S, 128), jnp.float32),
            pltpu.SemaphoreType.DMA,
            pltpu.SemaphoreType.DMA,
        ],
        compiler_params=pltpu.CompilerParams(use_tc_tiling_on_sc=True),
    )
    def k(idx_hbm, table_hbm, out_hbm, idx_v, rows_v, sem_g, sem_o):
        wid = lax.axis_index("s") * NC + lax.axis_index("c")
        brow0 = wid * rows_w

        def load_quarter(qi):
            pltpu.sync_copy(
                idx_hbm.at[pl.ds(brow0 + qi * qrows, qrows)], idx_v.at[qi % 2]
            )

        def start_gathers(g, b):
            h = (g // qchunks) % 2
            for r in range(RPC):
                rr = (g % qchunks) * RPC + r
                for o, n in zip(GRPS, GRPL):
                    pltpu.async_copy(
                        table_hbm.at[idx_v.at[h, rr, pl.ds(o, n)]],
                        rows_v.at[b, pl.ds(r * S + o, n)],
                        sem_g,
                    )

        def wait_chunk(b, sem):
            pltpu.make_async_copy(
                out_hbm.at[pl.ds(0, RPC * S)], rows_v.at[b], sem
            ).wait()

        load_quarter(0)
        start_gathers(0, 0)

        def chunk_body(g, carry):
            b = g % 2
            wait_chunk(b, sem_g)

            @pl.when(jnp.logical_and((g + 1) % qchunks == 0, g + 1 < n_chunks))
            def _():
                load_quarter((g + 1) // qchunks)

            @pl.when(g + 1 < n_chunks)
            def _():
                @pl.when(g >= 1)
                def _():
                    wait_chunk(1 - b, sem_o)
                start_gathers(g + 1, 1 - b)

            pltpu.async_copy(
                rows_v.at[b],
                out_hbm.at[pl.ds((brow0 + g * RPC) * S, RPC * S)],
                sem_o,
            )
            return carry

        lax.fori_loop(0, n_chunks, chunk_body, 0)
        wait_chunk(0, sem_o)
        wait_chunk(1, sem_o)

    return k


def _ln_body(e_ref, gb_ref, out_ref):
    x = e_ref[...][:, :64]
    gam = gb_ref[0, :]
    bet = gb_ref[1, :]
    inv_d = jnp.float32(1.0 / 64)
    s = jnp.sum(x, axis=1) * inv_d
    q = jnp.sum(x * x, axis=1) * inv_d
    r = lax.rsqrt(q - s * s + jnp.float32(EPS))
    nm = (x - s[:, None]) * r[:, None] * gam[None, :] + bet[None, :]
    nrows, S, D = out_ref.shape[2], out_ref.shape[0], out_ref.shape[1]
    out_ref[...] = jnp.transpose(nm.reshape(nrows, S, D), (1, 2, 0))


def _make_ln(B, S, D):
    return pl.pallas_call(
        _ln_body,
        grid=(B // BBT,),
        in_specs=[
            pl.BlockSpec((BBT * S, 128), lambda i: (i, 0)),
            pl.BlockSpec((2, D), lambda i: (0, 0)),
        ],
        out_specs=pl.BlockSpec((S, D, BBT), lambda i: (0, 0, i)),
        out_shape=jax.ShapeDtypeStruct((S, D, B), jnp.float32),
        compiler_params=pltpu.CompilerParams(
            dimension_semantics=("arbitrary",),
            vmem_limit_bytes=100 * 1024 * 1024,
        ),
    )


def kernel(x, table, gamma, beta):
    B, S = x.shape
    V, D = table.shape
    tp = jnp.pad(table, ((0, 0), (0, 128 - D)))
    e = _make_gather(B, S, V)(x.astype(jnp.int32), tp)
    gb = jnp.stack([gamma, beta]).astype(jnp.float32)
    out_t = _make_ln(B, S, D)(e, gb)
    return out_t.transpose(2, 0, 1)
